# Initial kernel scaffold; baseline (speedup 1.0000x reference)
import jax
import jax.numpy as jnp
from jax.experimental import pallas as pl


def kernel(x, edge_index, W1, b1, W2, b2, W3, b3, Wc1, bc1, Wc2, bc2):
    raise NotImplementedError("write your pallas kernel here")


# speedup vs baseline: 5.6982x; 5.6982x over previous
import functools

import jax
import jax.numpy as jnp
from jax import lax
from jax.experimental import pallas as pl
from jax.experimental.pallas import tpu as pltpu
from jax.experimental.pallas import tpu_sc as plsc

_LANES = 16
_NCORES = 2
_NSUB = 16
_CHUNK = 128


def _mesh():
    return plsc.VectorSubcoreMesh(core_axis_name="c", subcore_axis_name="s")


def _embed_body(x_ref, w1_ref, b1_ref, w2_ref, b2_ref, w3_ref, b3_ref, o_ref):
    P = x_ref.shape[1]
    acc = None
    for p in range(P):
        a = jnp.dot(x_ref[:, p, :], w1_ref[...],
                    preferred_element_type=jnp.float32)
        a = jnp.maximum(a + b1_ref[...], 0.0)
        acc = a if acc is None else acc + a
    h2 = jnp.dot(acc, w2_ref[...], preferred_element_type=jnp.float32)
    h2 = h2 + P * b2_ref[...]
    h3 = jnp.dot(h2, w3_ref[...], preferred_element_type=jnp.float32)
    h3 = h3 + b3_ref[...]
    o_ref[...] = jnp.maximum(h3, 0.0)


def _embed(x, W1, b1, W2, b2, W3, b3, bn=256):
    N, P, Cin = x.shape
    H = W1.shape[1]
    C2 = W2.shape[1]
    C3 = W3.shape[1]
    return pl.pallas_call(
        _embed_body,
        grid=(pl.cdiv(N, bn),),
        in_specs=[
            pl.BlockSpec((bn, P, Cin), lambda i: (i, 0, 0)),
            pl.BlockSpec((Cin, H), lambda i: (0, 0)),
            pl.BlockSpec((1, H), lambda i: (0, 0)),
            pl.BlockSpec((H, C2), lambda i: (0, 0)),
            pl.BlockSpec((1, C2), lambda i: (0, 0)),
            pl.BlockSpec((C2, C3), lambda i: (0, 0)),
            pl.BlockSpec((1, C3), lambda i: (0, 0)),
        ],
        out_specs=pl.BlockSpec((bn, C3), lambda i: (i, 0)),
        out_shape=jax.ShapeDtypeStruct((N, C3), jnp.float32),
    )(x, W1, b1.reshape(1, H), W2, b2.reshape(1, C2), W3, b3.reshape(1, C3))


_DEGW = 128


def _sc_deg(dst_pad, N_pad):
    E_pad = dst_pad.shape[0]
    per_sc = E_pad // _NCORES
    per_tile = per_sc // _NSUB
    n_chunks = per_tile // _CHUNK
    rw = N_pad // _NSUB

    @functools.partial(
        pl.kernel,
        mesh=_mesh(),
        out_type=jax.ShapeDtypeStruct((_NCORES, N_pad, _DEGW), jnp.float32),
        scratch_types=[
            pltpu.VMEM((_CHUNK,), jnp.int32),
            pltpu.VMEM((_CHUNK, _DEGW), jnp.float32),
            pltpu.VMEM_SHARED((N_pad, _DEGW), jnp.float32),
        ],
    )
    def deg_kernel(dst_hbm, zeros_hbm, ones_hbm, out_hbm, dst_v, ones_v, accd):
        c = lax.axis_index("c")
        s = lax.axis_index("s")
        pltpu.sync_copy(zeros_hbm.at[pl.ds(s * rw, rw)],
                        accd.at[pl.ds(s * rw, rw)])
        pltpu.sync_copy(ones_hbm, ones_v)
        plsc.subcore_barrier()
        base = c * per_sc + s * per_tile

        def chunk(j, carry):
            pltpu.sync_copy(dst_hbm.at[pl.ds(base + j * _CHUNK, _CHUNK)], dst_v)
            pltpu.sync_copy(ones_v, accd.at[dst_v], add=True)
            return carry

        lax.fori_loop(0, n_chunks, chunk, 0)
        plsc.subcore_barrier()
        pltpu.sync_copy(accd.at[pl.ds(s * rw, rw)],
                        out_hbm.at[c, pl.ds(s * rw, rw)])

    zeros = jnp.zeros((N_pad, _DEGW), jnp.float32)
    ones = jnp.ones((_CHUNK, _DEGW), jnp.float32)
    return deg_kernel(dst_pad, zeros, ones)


def _sc_conv(z_flat, src_pad, dst_pad, N_pad, Dh):
    E_pad = src_pad.shape[0]
    per_tile = E_pad // _NSUB
    n_chunks = per_tile // _CHUNK
    rw = N_pad // _NSUB

    @functools.partial(
        pl.kernel,
        mesh=_mesh(),
        out_type=jax.ShapeDtypeStruct((_NCORES, N_pad, Dh), jnp.float32),
        scratch_types=[
            pltpu.VMEM((_CHUNK,), jnp.int32),
            pltpu.VMEM((_CHUNK,), jnp.int32),
            pltpu.VMEM((_CHUNK, Dh), jnp.float32),
            pltpu.VMEM_SHARED((N_pad, Dh), jnp.float32),
            pltpu.SemaphoreType.DMA,
        ],
    )
    def conv_kernel(z_hbm, src_hbm, dst_hbm, out_hbm,
                    src_v, dst_v, rows_v, acc, sem):
        c = lax.axis_index("c")
        s = lax.axis_index("s")
        pltpu.sync_copy(z_hbm.at[pl.ds(c * N_pad + s * rw, rw)],
                        acc.at[pl.ds(s * rw, rw)])
        plsc.subcore_barrier()
        ebase = s * per_tile

        def chunk(j, carry):
            off = ebase + j * _CHUNK
            pltpu.sync_copy(src_hbm.at[pl.ds(off, _CHUNK)], src_v)
            pltpu.sync_copy(dst_hbm.at[pl.ds(off, _CHUNK)], dst_v)
            cN = c * N_pad
            for i in range(_CHUNK // _LANES):
                sl = pl.ds(i * _LANES, _LANES)
                src_v[sl] = src_v[sl] + cN
            pltpu.async_copy(z_hbm.at[src_v], rows_v, sem).wait()
            pltpu.sync_copy(rows_v, acc.at[dst_v], add=True)
            return carry

        lax.fori_loop(0, n_chunks, chunk, 0)
        plsc.subcore_barrier()
        pltpu.sync_copy(acc.at[pl.ds(s * rw, rw)],
                        out_hbm.at[c, pl.ds(s * rw, rw)])

    return conv_kernel(z_flat, src_pad, dst_pad)


def _dinv_of(deg_ref):
    deg = deg_ref[0, :, 0:1] + deg_ref[1, :, 0:1] + 1.0
    return lax.rsqrt(deg)


def _z1_body(h_ref, deg_ref, wc_ref, o_ref):
    Dh = o_ref.shape[2]
    dinv = _dinv_of(deg_ref)
    z = jnp.dot(h_ref[...], wc_ref[...],
                preferred_element_type=jnp.float32) * dinv
    o_ref[0] = z[:, :Dh]
    o_ref[1] = z[:, Dh:]


def _z1(h, degpair, Wc, N_pad, bn=2000):
    N, C = h.shape
    Dh = C // 2
    return pl.pallas_call(
        _z1_body,
        grid=(pl.cdiv(N, bn),),
        in_specs=[
            pl.BlockSpec((bn, C), lambda i: (i, 0)),
            pl.BlockSpec((2, bn, _DEGW), lambda i: (0, i, 0)),
            pl.BlockSpec((C, C), lambda i: (0, 0)),
        ],
        out_specs=pl.BlockSpec((2, bn, Dh), lambda i: (0, i, 0)),
        out_shape=jax.ShapeDtypeStruct((2, N_pad, Dh), jnp.float32),
    )(h, degpair, Wc)


def _z2_body(a1_ref, deg_ref, bc1_ref, wc2_ref, o_ref):
    Dh = o_ref.shape[2]
    dinv = _dinv_of(deg_ref)
    y0 = jnp.maximum(a1_ref[0] * dinv + bc1_ref[:, :Dh], 0.0)
    y1 = jnp.maximum(a1_ref[1] * dinv + bc1_ref[:, Dh:], 0.0)
    z = (jnp.dot(y0, wc2_ref[:Dh, :], preferred_element_type=jnp.float32)
         + jnp.dot(y1, wc2_ref[Dh:, :], preferred_element_type=jnp.float32))
    z = z * dinv
    o_ref[0] = z[:, :Dh]
    o_ref[1] = z[:, Dh:]


def _z2(a1, degpair, bc1, Wc2, N, bn=2000):
    _, N_pad, Dh = a1.shape
    C = 2 * Dh
    return pl.pallas_call(
        _z2_body,
        grid=(pl.cdiv(N, bn),),
        in_specs=[
            pl.BlockSpec((2, bn, Dh), lambda i: (0, i, 0)),
            pl.BlockSpec((2, bn, _DEGW), lambda i: (0, i, 0)),
            pl.BlockSpec((1, C), lambda i: (0, 0)),
            pl.BlockSpec((C, C), lambda i: (0, 0)),
        ],
        out_specs=pl.BlockSpec((2, bn, Dh), lambda i: (0, i, 0)),
        out_shape=jax.ShapeDtypeStruct((2, N_pad, Dh), jnp.float32),
    )(a1, degpair, bc1.reshape(1, C), Wc2)


def _final_body(a2_ref, deg_ref, bc2_ref, h_ref, o_ref):
    Dh = a2_ref.shape[2]
    dinv = _dinv_of(deg_ref)
    o20 = a2_ref[0] * dinv + bc2_ref[:, :Dh]
    o21 = a2_ref[1] * dinv + bc2_ref[:, Dh:]
    h0 = h_ref[:, :Dh]
    h1 = h_ref[:, Dh:]
    o_ref[:, :Dh] = jnp.maximum(h0 + o20, 0.0) + h0
    o_ref[:, Dh:] = jnp.maximum(h1 + o21, 0.0) + h1


def _final(a2, degpair, bc2, h, bn=2000):
    N, C = h.shape
    Dh = C // 2
    return pl.pallas_call(
        _final_body,
        grid=(pl.cdiv(N, bn),),
        in_specs=[
            pl.BlockSpec((2, bn, Dh), lambda i: (0, i, 0)),
            pl.BlockSpec((2, bn, _DEGW), lambda i: (0, i, 0)),
            pl.BlockSpec((1, C), lambda i: (0, 0)),
            pl.BlockSpec((bn, C), lambda i: (i, 0)),
        ],
        out_specs=pl.BlockSpec((bn, C), lambda i: (i, 0)),
        out_shape=jax.ShapeDtypeStruct((N, C), jnp.float32),
    )(a2, degpair, bc2.reshape(1, C), h)


def kernel(x, edge_index, W1, b1, W2, b2, W3, b3, Wc1, bc1, Wc2, bc2):
    N, P, Cin = x.shape
    C = W2.shape[1]
    Dh = C // 2
    E = edge_index.shape[1]

    align = _NCORES * _NSUB * _CHUNK
    E_pad = ((E + align - 1) // align) * align
    pad = E_pad - E
    nalign = _NSUB * 8
    N_pad = ((N + 1 + nalign - 1) // nalign) * nalign
    src_p = jnp.concatenate(
        [edge_index[0], jnp.zeros((pad,), edge_index.dtype)])
    dst_p = jnp.concatenate(
        [edge_index[1], jnp.full((pad,), N, edge_index.dtype)])

    h = _embed(x, W1, b1, W2, b2, W3, b3)
    degpair = _sc_deg(dst_p, N_pad)
    z1 = _z1(h, degpair, Wc1, N_pad)
    a1 = _sc_conv(z1.reshape(2 * N_pad, Dh), src_p, dst_p, N_pad, Dh)
    z2 = _z2(a1, degpair, bc1, Wc2, N)
    a2 = _sc_conv(z2.reshape(2 * N_pad, Dh), src_p, dst_p, N_pad, Dh)
    return _final(a2, degpair, bc2, h)


# speedup vs baseline: 8.8163x; 1.5472x over previous
import functools

import jax
import jax.numpy as jnp
from jax import lax
from jax.experimental import pallas as pl
from jax.experimental.pallas import tpu as pltpu
from jax.experimental.pallas import tpu_sc as plsc

_LANES = 16
_NCORES = 2
_NSUB = 16
_CHUNK = 128


def _mesh():
    return plsc.VectorSubcoreMesh(core_axis_name="c", subcore_axis_name="s")


def _embed_body(x_ref, w1_ref, b1_ref, w2_ref, b2_ref, w3_ref, b3_ref, o_ref):
    P = x_ref.shape[1]
    acc = None
    for p in range(P):
        a = jnp.dot(x_ref[:, p, :], w1_ref[...],
                    preferred_element_type=jnp.float32)
        a = jnp.maximum(a + b1_ref[...], 0.0)
        acc = a if acc is None else acc + a
    h2 = jnp.dot(acc, w2_ref[...], preferred_element_type=jnp.float32)
    h2 = h2 + P * b2_ref[...]
    h3 = jnp.dot(h2, w3_ref[...], preferred_element_type=jnp.float32)
    h3 = h3 + b3_ref[...]
    o_ref[...] = jnp.maximum(h3, 0.0)


def _embed(x, W1, b1, W2, b2, W3, b3, bn=256):
    N, P, Cin = x.shape
    H = W1.shape[1]
    C2 = W2.shape[1]
    C3 = W3.shape[1]
    return pl.pallas_call(
        _embed_body,
        grid=(pl.cdiv(N, bn),),
        in_specs=[
            pl.BlockSpec((bn, P, Cin), lambda i: (i, 0, 0)),
            pl.BlockSpec((Cin, H), lambda i: (0, 0)),
            pl.BlockSpec((1, H), lambda i: (0, 0)),
            pl.BlockSpec((H, C2), lambda i: (0, 0)),
            pl.BlockSpec((1, C2), lambda i: (0, 0)),
            pl.BlockSpec((C2, C3), lambda i: (0, 0)),
            pl.BlockSpec((1, C3), lambda i: (0, 0)),
        ],
        out_specs=pl.BlockSpec((bn, C3), lambda i: (i, 0)),
        out_shape=jax.ShapeDtypeStruct((N, C3), jnp.float32),
    )(x, W1, b1.reshape(1, H), W2, b2.reshape(1, C2), W3, b3.reshape(1, C3))


_DEGW = 128


def _sc_deg(dst2d, N_pad):
    total_rows = dst2d.shape[0]
    n_chunks = total_rows // (_NCORES * _NSUB)
    rw = N_pad // _NSUB

    @functools.partial(
        pl.kernel,
        mesh=_mesh(),
        out_type=jax.ShapeDtypeStruct((_NCORES, N_pad, _DEGW), jnp.float32),
        scratch_types=[
            pltpu.VMEM((n_chunks, _CHUNK), jnp.int32),
            pltpu.VMEM((_CHUNK, _DEGW), jnp.float32),
            pltpu.VMEM_SHARED((N_pad, _DEGW), jnp.float32),
        ],
    )
    def deg_kernel(dst_hbm, zeros_hbm, ones_hbm, out_hbm, dsts, ones_v, accd):
        c = lax.axis_index("c")
        s = lax.axis_index("s")
        pltpu.sync_copy(zeros_hbm.at[pl.ds(s * rw, rw)],
                        accd.at[pl.ds(s * rw, rw)])
        pltpu.sync_copy(ones_hbm, ones_v)
        rbase = c * (_NSUB * n_chunks) + s * n_chunks
        pltpu.sync_copy(dst_hbm.at[pl.ds(rbase, n_chunks)], dsts)
        plsc.subcore_barrier()

        def chunk(j, carry):
            pltpu.sync_copy(ones_v, accd.at[dsts.at[j]], add=True)
            return carry

        lax.fori_loop(0, n_chunks, chunk, 0)
        plsc.subcore_barrier()
        pltpu.sync_copy(accd.at[pl.ds(s * rw, rw)],
                        out_hbm.at[c, pl.ds(s * rw, rw)])

    zeros = jnp.zeros((N_pad, _DEGW), jnp.float32)
    ones = jnp.ones((_CHUNK, _DEGW), jnp.float32)
    return deg_kernel(dst2d, zeros, ones)


def _sc_conv(z_flat, src2d, dst2d, N_pad, Dh):
    total_rows = src2d.shape[0]
    rows_pt = total_rows // _NSUB
    n_half = 2
    half = rows_pt // n_half
    assert half % 2 == 0
    rw = N_pad // _NSUB

    @functools.partial(
        pl.kernel,
        mesh=_mesh(),
        out_type=jax.ShapeDtypeStruct((_NCORES, N_pad, Dh), jnp.float32),
        scratch_types=[
            pltpu.VMEM((half, _CHUNK), jnp.int32),
            pltpu.VMEM((half, _CHUNK), jnp.int32),
            pltpu.VMEM((_CHUNK, Dh), jnp.float32),
            pltpu.VMEM((_CHUNK, Dh), jnp.float32),
            pltpu.VMEM_SHARED((N_pad, Dh), jnp.float32),
            pltpu.SemaphoreType.DMA,
            pltpu.SemaphoreType.DMA,
        ],
    )
    def conv_kernel(z_hbm, src_hbm, dst_hbm, out_hbm,
                    srcs, dsts, rows0, rows1, acc, sem0, sem1):
        c = lax.axis_index("c")
        s = lax.axis_index("s")
        pltpu.sync_copy(z_hbm.at[pl.ds(c * N_pad + s * rw, rw)],
                        acc.at[pl.ds(s * rw, rw)])
        plsc.subcore_barrier()
        cN = c * N_pad

        def fix(j, carry):
            for i in range(_CHUNK // _LANES):
                sl = pl.ds(i * _LANES, _LANES)
                srcs[j, sl] = srcs[j, sl] + cN
            return carry

        def pair(k, carry):
            j0 = 2 * k
            pltpu.async_copy(z_hbm.at[srcs.at[j0 + 1]], rows1, sem1)
            pltpu.make_async_copy(z_hbm.at[srcs.at[j0]], rows0, sem0).wait()
            pltpu.sync_copy(rows0, acc.at[dsts.at[j0]], add=True)

            @pl.when(k + 1 < half // 2)
            def _():
                pltpu.async_copy(z_hbm.at[srcs.at[j0 + 2]], rows0, sem0)

            pltpu.make_async_copy(
                z_hbm.at[srcs.at[j0 + 1]], rows1, sem1).wait()
            pltpu.sync_copy(rows1, acc.at[dsts.at[j0 + 1]], add=True)
            return carry

        for hh in range(n_half):
            rbase = s * rows_pt + hh * half
            pltpu.sync_copy(src_hbm.at[pl.ds(rbase, half)], srcs)
            pltpu.sync_copy(dst_hbm.at[pl.ds(rbase, half)], dsts)
            lax.fori_loop(0, half, fix, 0)
            pltpu.async_copy(z_hbm.at[srcs.at[0]], rows0, sem0)
            lax.fori_loop(0, half // 2, pair, 0)

        plsc.subcore_barrier()
        pltpu.sync_copy(acc.at[pl.ds(s * rw, rw)],
                        out_hbm.at[c, pl.ds(s * rw, rw)])

    return conv_kernel(z_flat, src2d, dst2d)


def _dinv_of(deg_ref):
    deg = deg_ref[0, :, 0:1] + deg_ref[1, :, 0:1] + 1.0
    return lax.rsqrt(deg)


def _z1_body(h_ref, deg_ref, wc_ref, o_ref):
    Dh = o_ref.shape[2]
    dinv = _dinv_of(deg_ref)
    z = jnp.dot(h_ref[...], wc_ref[...],
                preferred_element_type=jnp.float32) * dinv
    o_ref[0] = z[:, :Dh]
    o_ref[1] = z[:, Dh:]


def _z1(h, degpair, Wc, N_pad, bn=2000):
    N, C = h.shape
    Dh = C // 2
    return pl.pallas_call(
        _z1_body,
        grid=(pl.cdiv(N, bn),),
        in_specs=[
            pl.BlockSpec((bn, C), lambda i: (i, 0)),
            pl.BlockSpec((2, bn, _DEGW), lambda i: (0, i, 0)),
            pl.BlockSpec((C, C), lambda i: (0, 0)),
        ],
        out_specs=pl.BlockSpec((2, bn, Dh), lambda i: (0, i, 0)),
        out_shape=jax.ShapeDtypeStruct((2, N_pad, Dh), jnp.float32),
    )(h, degpair, Wc)


def _z2_body(a1_ref, deg_ref, bc1_ref, wc2_ref, o_ref):
    Dh = o_ref.shape[2]
    dinv = _dinv_of(deg_ref)
    y0 = jnp.maximum(a1_ref[0] * dinv + bc1_ref[:, :Dh], 0.0)
    y1 = jnp.maximum(a1_ref[1] * dinv + bc1_ref[:, Dh:], 0.0)
    z = (jnp.dot(y0, wc2_ref[:Dh, :], preferred_element_type=jnp.float32)
         + jnp.dot(y1, wc2_ref[Dh:, :], preferred_element_type=jnp.float32))
    z = z * dinv
    o_ref[0] = z[:, :Dh]
    o_ref[1] = z[:, Dh:]


def _z2(a1, degpair, bc1, Wc2, N, bn=2000):
    _, N_pad, Dh = a1.shape
    C = 2 * Dh
    return pl.pallas_call(
        _z2_body,
        grid=(pl.cdiv(N, bn),),
        in_specs=[
            pl.BlockSpec((2, bn, Dh), lambda i: (0, i, 0)),
            pl.BlockSpec((2, bn, _DEGW), lambda i: (0, i, 0)),
            pl.BlockSpec((1, C), lambda i: (0, 0)),
            pl.BlockSpec((C, C), lambda i: (0, 0)),
        ],
        out_specs=pl.BlockSpec((2, bn, Dh), lambda i: (0, i, 0)),
        out_shape=jax.ShapeDtypeStruct((2, N_pad, Dh), jnp.float32),
    )(a1, degpair, bc1.reshape(1, C), Wc2)


def _final_body(a2_ref, deg_ref, bc2_ref, h_ref, o_ref):
    Dh = a2_ref.shape[2]
    dinv = _dinv_of(deg_ref)
    o20 = a2_ref[0] * dinv + bc2_ref[:, :Dh]
    o21 = a2_ref[1] * dinv + bc2_ref[:, Dh:]
    h0 = h_ref[:, :Dh]
    h1 = h_ref[:, Dh:]
    o_ref[:, :Dh] = jnp.maximum(h0 + o20, 0.0) + h0
    o_ref[:, Dh:] = jnp.maximum(h1 + o21, 0.0) + h1


def _final(a2, degpair, bc2, h, bn=2000):
    N, C = h.shape
    Dh = C // 2
    return pl.pallas_call(
        _final_body,
        grid=(pl.cdiv(N, bn),),
        in_specs=[
            pl.BlockSpec((2, bn, Dh), lambda i: (0, i, 0)),
            pl.BlockSpec((2, bn, _DEGW), lambda i: (0, i, 0)),
            pl.BlockSpec((1, C), lambda i: (0, 0)),
            pl.BlockSpec((bn, C), lambda i: (i, 0)),
        ],
        out_specs=pl.BlockSpec((bn, C), lambda i: (i, 0)),
        out_shape=jax.ShapeDtypeStruct((N, C), jnp.float32),
    )(a2, degpair, bc2.reshape(1, C), h)


def kernel(x, edge_index, W1, b1, W2, b2, W3, b3, Wc1, bc1, Wc2, bc2):
    N, P, Cin = x.shape
    C = W2.shape[1]
    Dh = C // 2
    E = edge_index.shape[1]

    align = _NCORES * _NSUB * _CHUNK
    E_pad = ((E + align - 1) // align) * align
    pad = E_pad - E
    nalign = _NSUB * 8
    N_pad = ((N + 1 + nalign - 1) // nalign) * nalign
    src2d = jnp.concatenate(
        [edge_index[0], jnp.zeros((pad,), edge_index.dtype)]
    ).reshape(E_pad // _CHUNK, _CHUNK)
    dst2d = jnp.concatenate(
        [edge_index[1], jnp.full((pad,), N, edge_index.dtype)]
    ).reshape(E_pad // _CHUNK, _CHUNK)

    h = _embed(x, W1, b1, W2, b2, W3, b3)
    degpair = _sc_deg(dst2d, N_pad)
    z1 = _z1(h, degpair, Wc1, N_pad)
    a1 = _sc_conv(z1.reshape(2 * N_pad, Dh), src2d, dst2d, N_pad, Dh)
    z2 = _z2(a1, degpair, bc1, Wc2, N)
    a2 = _sc_conv(z2.reshape(2 * N_pad, Dh), src2d, dst2d, N_pad, Dh)
    return _final(a2, degpair, bc2, h)
